# baseline (speedup 1.0000x reference)
import functools

import jax
import jax.numpy as jnp
from jax import lax
from jax.experimental import pallas as pl
from jax.experimental.pallas import tpu as pltpu
from jax.experimental.pallas import tpu_sc as plsc

_B = 16384
_D = 32
_NC = 2
_NS = 16
_NW = _NC * _NS
_BPW = _B // _NW
_CHUNK = 128
_NCHUNK = _BPW // _CHUNK

_TC_BLOCK = 2048


_NROW = _D * _NCHUNK


def _sc_gather_body(uoff_hbm, ioff_hbm, uflat, iflat,
                    uT_out, iT_out,
                    uoff_v, ioff_v, ubuf, ibuf, sem):
    wid = lax.axis_index("s") * _NC + lax.axis_index("c")
    base = wid * _BPW
    pltpu.sync_copy(uoff_hbm.at[wid], uoff_v)
    pltpu.sync_copy(ioff_hbm.at[wid], ioff_v)

    def body(r, carry):
        d = r // _NCHUNK
        c = r % _NCHUNK
        pltpu.async_copy(uflat.at[uoff_v.at[r]],
                         ubuf.at[d, pl.ds(c * _CHUNK, _CHUNK)], sem)
        pltpu.async_copy(iflat.at[ioff_v.at[r]],
                         ibuf.at[d, pl.ds(c * _CHUNK, _CHUNK)], sem)
        return carry

    lax.fori_loop(0, _NROW, body, 0)
    pltpu.make_async_copy(uT_out.at[:, pl.ds(base, _BPW)], ubuf, sem).wait()
    pltpu.make_async_copy(iT_out.at[:, pl.ds(base, _BPW)], ibuf, sem).wait()
    pltpu.sync_copy(ubuf, uT_out.at[:, pl.ds(base, _BPW)])
    pltpu.sync_copy(ibuf, iT_out.at[:, pl.ds(base, _BPW)])


def _sc_gather(user_idx, item_idx, user_emb, item_emb):
    uflat = user_emb.T.reshape(-1)
    iflat = item_emb.T.reshape(-1)
    V = user_emb.shape[0]
    dshift = (jnp.arange(_D, dtype=jnp.int32) * V).reshape(1, _D, 1, 1)
    uoff = (user_idx.reshape(_NW, 1, _NCHUNK, _CHUNK) + dshift
            ).reshape(_NW, _NROW, _CHUNK)
    ioff = (item_idx.reshape(_NW, 1, _NCHUNK, _CHUNK) + dshift
            ).reshape(_NW, _NROW, _CHUNK)
    mesh = plsc.VectorSubcoreMesh(core_axis_name="c", subcore_axis_name="s")
    f = pl.kernel(
        _sc_gather_body,
        mesh=mesh,
        out_type=(
            jax.ShapeDtypeStruct((_D, _B), jnp.float32),
            jax.ShapeDtypeStruct((_D, _B), jnp.float32),
        ),
        scratch_types=[
            pltpu.VMEM((_NROW, _CHUNK), jnp.int32),
            pltpu.VMEM((_NROW, _CHUNK), jnp.int32),
            pltpu.VMEM((_D, _BPW), jnp.float32),
            pltpu.VMEM((_D, _BPW), jnp.float32),
            pltpu.SemaphoreType.DMA,
        ],
    )
    return f(uoff, ioff, uflat, iflat)


def _tc_body(uT_ref, iT_ref, dnT_ref, WdT_ref, bd_ref,
             W1uT_ref, W1iT_ref, W1dT_ref, b1_ref,
             W2T_ref, b2_ref, W3T_ref, b3_ref, out_ref):
    uT = uT_ref[...]
    iT = iT_ref[...]
    dnT = dnT_ref[...]
    dT = jnp.dot(WdT_ref[...], dnT, preferred_element_type=jnp.float32) + bd_ref[...]
    s = uT + iT + dT
    fm = 0.5 * jnp.sum(s * s - uT * uT - iT * iT - dT * dT,
                       axis=0, keepdims=True)
    hT = (jnp.dot(W1uT_ref[...], uT, preferred_element_type=jnp.float32)
          + jnp.dot(W1iT_ref[...], iT, preferred_element_type=jnp.float32)
          + jnp.dot(W1dT_ref[...], dnT, preferred_element_type=jnp.float32)
          + b1_ref[...])
    hT = jnp.maximum(hT, 0.0)
    hT = jnp.maximum(
        jnp.dot(W2T_ref[...], hT, preferred_element_type=jnp.float32) + b2_ref[...],
        0.0)
    outT = jnp.dot(W3T_ref[...], hT, preferred_element_type=jnp.float32) + b3_ref[...]
    out_ref[...] = outT + fm


def _tc_compute(uT, iT, dense, Wd, bd, W1, b1, W2, b2, W3, b3):
    nd = dense.shape[1]
    h1 = W1.shape[1]
    h2 = W2.shape[1]
    dnT = dense.T
    WdT = Wd.T
    W1uT = W1[:_D].T
    W1iT = W1[_D:2 * _D].T
    W1dT = W1[2 * _D:].T
    W2T = W2.T
    W3T = W3.T
    grid = _B // _TC_BLOCK

    def batch_spec(rows):
        return pl.BlockSpec((rows, _TC_BLOCK), lambda b: (0, b))

    def full_spec(shape):
        return pl.BlockSpec(shape, lambda b: (0,) * len(shape))

    out = pl.pallas_call(
        _tc_body,
        grid=(grid,),
        in_specs=[
            batch_spec(_D), batch_spec(_D), batch_spec(nd),
            full_spec(WdT.shape), full_spec((_D, 1)),
            full_spec(W1uT.shape), full_spec(W1iT.shape), full_spec(W1dT.shape),
            full_spec((h1, 1)),
            full_spec(W2T.shape), full_spec((h2, 1)),
            full_spec(W3T.shape), full_spec((1, 1)),
        ],
        out_specs=pl.BlockSpec((1, _TC_BLOCK), lambda b: (0, b)),
        out_shape=jax.ShapeDtypeStruct((1, _B), jnp.float32),
    )(uT, iT, dnT, WdT, bd.reshape(_D, 1),
      W1uT, W1iT, W1dT, b1.reshape(h1, 1),
      W2T, b2.reshape(h2, 1), W3T, b3.reshape(1, 1))
    return out[0]


def kernel(user_idx, item_idx, dense, user_emb, item_emb,
           Wd, bd, W1, b1, W2, b2, W3, b3):
    uT, iT = _sc_gather(user_idx, item_idx, user_emb, item_emb)
    return _tc_compute(uT, iT, dense, Wd, bd, W1, b1, W2, b2, W3, b3)


# speedup vs baseline: 5.4714x; 5.4714x over previous
import functools

import jax
import jax.numpy as jnp
from jax import lax
from jax.experimental import pallas as pl
from jax.experimental.pallas import tpu as pltpu
from jax.experimental.pallas import tpu_sc as plsc

_B = 16384
_D = 32
_PD = 128
_NC = 2
_NS = 16
_NW = _NC * _NS
_BPW = _B // _NW
_CHUNK = 128
_NCHUNK = _BPW // _CHUNK

_TC_BLOCK = 2048


def _sc_gather_body(uidx_hbm, iidx_hbm, utab, itab,
                    u_out, i_out,
                    uidx_v, iidx_v, ubuf, ibuf, sem):
    wid = lax.axis_index("s") * _NC + lax.axis_index("c")
    base = wid * _BPW
    pltpu.sync_copy(uidx_hbm.at[pl.ds(wid * _NCHUNK, _NCHUNK)], uidx_v)
    pltpu.sync_copy(iidx_hbm.at[pl.ds(wid * _NCHUNK, _NCHUNK)], iidx_v)
    half = _NCHUNK // 2
    for p in range(2):
        copies = []
        for c in range(half):
            copies.append(pltpu.async_copy(
                utab.at[uidx_v.at[p * half + c]],
                ubuf.at[pl.ds(c * _CHUNK, _CHUNK)], sem))
            copies.append(pltpu.async_copy(
                itab.at[iidx_v.at[p * half + c]],
                ibuf.at[pl.ds(c * _CHUNK, _CHUNK)], sem))
        for cp in copies:
            cp.wait()
        pltpu.sync_copy(ubuf, u_out.at[pl.ds(base + p * half * _CHUNK,
                                             half * _CHUNK)])
        pltpu.sync_copy(ibuf, i_out.at[pl.ds(base + p * half * _CHUNK,
                                             half * _CHUNK)])


def _sc_gather(user_idx, item_idx, utab_pad, itab_pad):
    mesh = plsc.VectorSubcoreMesh(core_axis_name="c", subcore_axis_name="s")
    f = pl.kernel(
        _sc_gather_body,
        mesh=mesh,
        out_type=(
            jax.ShapeDtypeStruct((_B, _PD), jnp.float32),
            jax.ShapeDtypeStruct((_B, _PD), jnp.float32),
        ),
        scratch_types=[
            pltpu.VMEM((_NCHUNK, _CHUNK), jnp.int32),
            pltpu.VMEM((_NCHUNK, _CHUNK), jnp.int32),
            pltpu.VMEM((_BPW // 2, _PD), jnp.float32),
            pltpu.VMEM((_BPW // 2, _PD), jnp.float32),
            pltpu.SemaphoreType.DMA,
        ],
    )
    uidx2 = user_idx.reshape(_NW * _NCHUNK, _CHUNK)
    iidx2 = item_idx.reshape(_NW * _NCHUNK, _CHUNK)
    return f(uidx2, iidx2, utab_pad, itab_pad)


def _tc_body(u_ref, i_ref, dn_ref, Wd_ref, bd_ref,
             W1u_ref, W1i_ref, W1d_ref, b1_ref,
             W2_ref, b2_ref, W3_ref, b3_ref, out_ref):
    u = u_ref[:, :_D]
    it = i_ref[:, :_D]
    dn = dn_ref[...]
    d = jnp.dot(dn, Wd_ref[...], preferred_element_type=jnp.float32) + bd_ref[...]
    s = u + it + d
    fm = 0.5 * jnp.sum(s * s - u * u - it * it - d * d, axis=1, keepdims=True)
    h = (jnp.dot(u, W1u_ref[...], preferred_element_type=jnp.float32)
         + jnp.dot(it, W1i_ref[...], preferred_element_type=jnp.float32)
         + jnp.dot(dn, W1d_ref[...], preferred_element_type=jnp.float32)
         + b1_ref[...])
    h = jnp.maximum(h, 0.0)
    h = jnp.maximum(
        jnp.dot(h, W2_ref[...], preferred_element_type=jnp.float32) + b2_ref[...],
        0.0)
    out = jnp.dot(h, W3_ref[...], preferred_element_type=jnp.float32) + b3_ref[...]
    out_ref[...] = out + fm


def _tc_compute(u, i, dense, Wd, bd, W1, b1, W2, b2, W3, b3):
    nd = dense.shape[1]
    h1 = W1.shape[1]
    h2 = W2.shape[1]
    W1u = W1[:_D]
    W1i = W1[_D:2 * _D]
    W1d = W1[2 * _D:]
    grid = _B // _TC_BLOCK

    def batch_spec(cols):
        return pl.BlockSpec((_TC_BLOCK, cols), lambda b: (b, 0))

    def full_spec(shape):
        return pl.BlockSpec(shape, lambda b: (0,) * len(shape))

    out = pl.pallas_call(
        _tc_body,
        grid=(grid,),
        in_specs=[
            batch_spec(_PD), batch_spec(_PD), batch_spec(nd),
            full_spec(Wd.shape), full_spec((1, _D)),
            full_spec(W1u.shape), full_spec(W1i.shape), full_spec(W1d.shape),
            full_spec((1, h1)),
            full_spec(W2.shape), full_spec((1, h2)),
            full_spec(W3.shape), full_spec((1, 1)),
        ],
        out_specs=pl.BlockSpec((_TC_BLOCK, 1), lambda b: (b, 0)),
        out_shape=jax.ShapeDtypeStruct((_B, 1), jnp.float32),
    )(u, i, dense, Wd, bd.reshape(1, _D),
      W1u, W1i, W1d, b1.reshape(1, h1),
      W2, b2.reshape(1, h2), W3, b3.reshape(1, 1))
    return out[:, 0]


def kernel(user_idx, item_idx, dense, user_emb, item_emb,
           Wd, bd, W1, b1, W2, b2, W3, b3):
    utab_pad = jnp.pad(user_emb, ((0, 0), (0, _PD - _D)))
    itab_pad = jnp.pad(item_emb, ((0, 0), (0, _PD - _D)))
    u, i = _sc_gather(user_idx, item_idx, utab_pad, itab_pad)
    return _tc_compute(u, i, dense, Wd, bd, W1, b1, W2, b2, W3, b3)
